# baseline (speedup 1.0000x reference)
import jax
import jax.numpy as jnp
from jax.experimental import pallas as pl

_K = 8192
_D = 32
_M_BLK = 2048
_K_CHUNK = 1024


def _vq_argmin_kernel(x_ref, ct_ref, out_ref):
    x = x_ref[...]
    x_sq = jnp.sum(x * x, axis=1, keepdims=True)
    best_val = jnp.full((_M_BLK, 1), jnp.inf, jnp.float32)
    best_idx = jnp.zeros((_M_BLK, 1), jnp.int32)
    for k0 in range(0, _K, _K_CHUNK):
        ct = ct_ref[:, k0:k0 + _K_CHUNK]
        c_sq = jnp.sum(ct * ct, axis=0, keepdims=True)
        mm = jax.lax.dot_general(
            x, ct, (((1,), (0,)), ((), ())),
            preferred_element_type=jnp.float32)
        l2 = (c_sq + x_sq) - 2.0 * mm
        min_val = jnp.min(l2, axis=1, keepdims=True)
        iota = jax.lax.broadcasted_iota(jnp.int32, l2.shape, 1) + k0
        idx = jnp.min(jnp.where(l2 == min_val, iota, _K), axis=1, keepdims=True)
        take = min_val < best_val
        best_val = jnp.where(take, min_val, best_val)
        best_idx = jnp.where(take, idx, best_idx)
    out_ref[...] = best_idx


def kernel(z_e_x, codebook):
    n, d, h, w = z_e_x.shape
    m = n * h * w
    flat = jnp.transpose(z_e_x, (0, 2, 3, 1)).reshape(m, d)
    ct = codebook.T
    grid = m // _M_BLK
    out = pl.pallas_call(
        _vq_argmin_kernel,
        grid=(grid,),
        in_specs=[
            pl.BlockSpec((_M_BLK, _D), lambda i: (i, 0)),
            pl.BlockSpec((_D, _K), lambda i: (0, 0)),
        ],
        out_specs=pl.BlockSpec((_M_BLK, 1), lambda i: (i, 0)),
        out_shape=jax.ShapeDtypeStruct((m, 1), jnp.int32),
    )(flat, ct)
    return out.reshape(n, h, w)


# speedup vs baseline: 1.0334x; 1.0334x over previous
import jax
import jax.numpy as jnp
from jax.experimental import pallas as pl

_K = 8192
_D = 32
_M_BLK = 1024
_K_CHUNK = 4096


def _vq_argmin_kernel(x_ref, ct_ref, out_ref):
    x = x_ref[...]
    x_sq = jnp.sum(x * x, axis=1, keepdims=True)
    best_val = jnp.full((_M_BLK, 1), jnp.inf, jnp.float32)
    best_idx = jnp.zeros((_M_BLK, 1), jnp.int32)
    for k0 in range(0, _K, _K_CHUNK):
        ct = ct_ref[:, k0:k0 + _K_CHUNK]
        c_sq = jnp.sum(ct * ct, axis=0, keepdims=True)
        mm = jax.lax.dot_general(
            x, ct, (((1,), (0,)), ((), ())),
            preferred_element_type=jnp.float32)
        l2 = (c_sq + x_sq) - 2.0 * mm
        min_val = jnp.min(l2, axis=1, keepdims=True)
        iota = jax.lax.broadcasted_iota(jnp.int32, l2.shape, 1) + k0
        idx = jnp.min(jnp.where(l2 == min_val, iota, _K), axis=1, keepdims=True)
        take = min_val < best_val
        best_val = jnp.where(take, min_val, best_val)
        best_idx = jnp.where(take, idx, best_idx)
    out_ref[...] = best_idx


def kernel(z_e_x, codebook):
    n, d, h, w = z_e_x.shape
    m = n * h * w
    flat = jnp.transpose(z_e_x, (0, 2, 3, 1)).reshape(m, d)
    ct = codebook.T
    grid = m // _M_BLK
    out = pl.pallas_call(
        _vq_argmin_kernel,
        grid=(grid,),
        in_specs=[
            pl.BlockSpec((_M_BLK, _D), lambda i: (i, 0)),
            pl.BlockSpec((_D, _K), lambda i: (0, 0)),
        ],
        out_specs=pl.BlockSpec((_M_BLK, 1), lambda i: (i, 0)),
        out_shape=jax.ShapeDtypeStruct((m, 1), jnp.int32),
    )(flat, ct)
    return out.reshape(n, h, w)


# speedup vs baseline: 1.0580x; 1.0238x over previous
import jax
import jax.numpy as jnp
from jax.experimental import pallas as pl

_K = 8192
_D = 32
_M_BLK = 512
_K_CHUNK = 8192


def _vq_argmin_kernel(x_ref, ct_ref, out_ref):
    x = x_ref[...]
    x_sq = jnp.sum(x * x, axis=1, keepdims=True)
    best_val = jnp.full((_M_BLK, 1), jnp.inf, jnp.float32)
    best_idx = jnp.zeros((_M_BLK, 1), jnp.int32)
    for k0 in range(0, _K, _K_CHUNK):
        ct = ct_ref[:, k0:k0 + _K_CHUNK]
        c_sq = jnp.sum(ct * ct, axis=0, keepdims=True)
        mm = jax.lax.dot_general(
            x, ct, (((1,), (0,)), ((), ())),
            preferred_element_type=jnp.float32)
        l2 = (c_sq + x_sq) - 2.0 * mm
        min_val = jnp.min(l2, axis=1, keepdims=True)
        iota = jax.lax.broadcasted_iota(jnp.int32, l2.shape, 1) + k0
        idx = jnp.min(jnp.where(l2 == min_val, iota, _K), axis=1, keepdims=True)
        take = min_val < best_val
        best_val = jnp.where(take, min_val, best_val)
        best_idx = jnp.where(take, idx, best_idx)
    out_ref[...] = best_idx


def kernel(z_e_x, codebook):
    n, d, h, w = z_e_x.shape
    m = n * h * w
    flat = jnp.transpose(z_e_x, (0, 2, 3, 1)).reshape(m, d)
    ct = codebook.T
    grid = m // _M_BLK
    out = pl.pallas_call(
        _vq_argmin_kernel,
        grid=(grid,),
        in_specs=[
            pl.BlockSpec((_M_BLK, _D), lambda i: (i, 0)),
            pl.BlockSpec((_D, _K), lambda i: (0, 0)),
        ],
        out_specs=pl.BlockSpec((_M_BLK, 1), lambda i: (i, 0)),
        out_shape=jax.ShapeDtypeStruct((m, 1), jnp.int32),
    )(flat, ct)
    return out.reshape(n, h, w)


# speedup vs baseline: 1.0915x; 1.0317x over previous
import jax
import jax.numpy as jnp
from jax.experimental import pallas as pl

_K = 8192
_D = 32
_M_BLK = 1024
_K_CHUNK = 8192


def _vq_argmin_kernel(x_ref, ct_ref, out_ref):
    x = x_ref[...]
    x_sq = jnp.sum(x * x, axis=1, keepdims=True)
    best_val = jnp.full((_M_BLK, 1), jnp.inf, jnp.float32)
    best_idx = jnp.zeros((_M_BLK, 1), jnp.int32)
    for k0 in range(0, _K, _K_CHUNK):
        ct = ct_ref[:, k0:k0 + _K_CHUNK]
        c_sq = jnp.sum(ct * ct, axis=0, keepdims=True)
        mm = jax.lax.dot_general(
            x, ct, (((1,), (0,)), ((), ())),
            preferred_element_type=jnp.float32)
        l2 = (c_sq + x_sq) - 2.0 * mm
        min_val = jnp.min(l2, axis=1, keepdims=True)
        iota = jax.lax.broadcasted_iota(jnp.int32, l2.shape, 1) + k0
        idx = jnp.min(jnp.where(l2 == min_val, iota, _K), axis=1, keepdims=True)
        take = min_val < best_val
        best_val = jnp.where(take, min_val, best_val)
        best_idx = jnp.where(take, idx, best_idx)
    out_ref[...] = best_idx


def kernel(z_e_x, codebook):
    n, d, h, w = z_e_x.shape
    m = n * h * w
    flat = jnp.transpose(z_e_x, (0, 2, 3, 1)).reshape(m, d)
    ct = codebook.T
    grid = m // _M_BLK
    out = pl.pallas_call(
        _vq_argmin_kernel,
        grid=(grid,),
        in_specs=[
            pl.BlockSpec((_M_BLK, _D), lambda i: (i, 0)),
            pl.BlockSpec((_D, _K), lambda i: (0, 0)),
        ],
        out_specs=pl.BlockSpec((_M_BLK, 1), lambda i: (i, 0)),
        out_shape=jax.ShapeDtypeStruct((m, 1), jnp.int32),
    )(flat, ct)
    return out.reshape(n, h, w)
